# baseline (speedup 1.0000x reference)
import functools

import jax
import jax.numpy as jnp
from jax import lax
from jax.experimental import pallas as pl
from jax.experimental.pallas import tpu as pltpu
from jax.experimental.pallas import tpu_sc as plsc

HID = 768
EPS = 1e-12


def _build_gather(total_tokens, nw, ch):
    tok_per_w = total_tokens // nw
    nchunks = tok_per_w // ch
    mesh = plsc.VectorSubcoreMesh(core_axis_name="c", subcore_axis_name="s")

    @functools.partial(
        pl.kernel,
        mesh=mesh,
        compiler_params=pltpu.CompilerParams(needs_layout_passes=False),
        out_type=jax.ShapeDtypeStruct((total_tokens, HID), jnp.float32),
        scratch_types=[
            pltpu.VMEM((tok_per_w,), jnp.int32),
            pltpu.VMEM((4, ch, HID), jnp.float32),
            pltpu.SemaphoreType.DMA,
            pltpu.SemaphoreType.DMA,
            pltpu.SemaphoreType.DMA,
            pltpu.SemaphoreType.DMA,
            pltpu.SemaphoreType.DMA,
            pltpu.SemaphoreType.DMA,
            pltpu.SemaphoreType.DMA,
            pltpu.SemaphoreType.DMA,
        ],
    )
    def gather_kernel(ids_hbm, wemb_hbm, raw_out, idx_v, a_v,
                      sg0, sg1, sg2, sg3, sw0, sw1, sw2, sw3):
        wid = lax.axis_index("s") * 2 + lax.axis_index("c")
        w0 = wid * tok_per_w
        sg = (sg0, sg1, sg2, sg3)
        sw = (sw0, sw1, sw2, sw3)

        pltpu.sync_copy(ids_hbm.at[pl.ds(w0, tok_per_w)], idx_v)

        def gather(cix, b):
            return pltpu.make_async_copy(
                wemb_hbm.at[idx_v.at[pl.ds(cix * ch, ch)]], a_v.at[b], sg[b])

        def write(cix, b):
            return pltpu.make_async_copy(
                a_v.at[b], raw_out.at[pl.ds(w0 + cix * ch, ch)], sw[b])

        for b in range(4):
            gather(b, b).start()

        def step(k, _):
            for b in range(4):
                cix = 4 * k + b
                gather(cix, b).wait()
                write(cix, b).start()

                @pl.when(k < nchunks // 4 - 1)
                def _():
                    write(cix, b).wait()
                    gather(cix + 4, b).start()

                @pl.when(k == nchunks // 4 - 1)
                def _():
                    write(cix, b).wait()
            return 0

        lax.fori_loop(0, nchunks // 4, step, 0)

    return gather_kernel


def _ln_block(raw_ref, pos_ref, tt_ref, temb_ref, gamma_ref, beta_ref,
              emb_ref):
    pos = pos_ref[...]
    t0 = temb_ref[0, :][None, None, :]
    td = (temb_ref[1, :] - temb_ref[0, :])[None, None, :]
    ttf = tt_ref[0, 0].astype(jnp.float32)
    x = (raw_ref[...] + pos[None, :, :]
         + t0 + ttf[:, :, None] * td)
    mean = jnp.mean(x, axis=-1, keepdims=True)
    xc = x - mean
    var = jnp.mean(xc * xc, axis=-1, keepdims=True)
    y = xc * lax.rsqrt(var + EPS)
    emb_ref[...] = y * gamma_ref[0][None, None, :] + beta_ref[0][None, None, :]


def _ln_tc(raw3, pos_emb, tt3, type_emb, gamma, beta, bsz, seq_len, sb):
    nblk = seq_len // sb
    return pl.pallas_call(
        _ln_block,
        grid=(nblk, bsz),
        in_specs=[
            pl.BlockSpec((1, sb, HID), lambda i, b: (b, i, 0)),
            pl.BlockSpec((sb, HID), lambda i, b: (i, 0)),
            pl.BlockSpec((1, 1, 1, sb), lambda i, b: (i, b, 0, 0)),
            pl.BlockSpec((2, HID), lambda i, b: (0, 0)),
            pl.BlockSpec((1, HID), lambda i, b: (0, 0)),
            pl.BlockSpec((1, HID), lambda i, b: (0, 0)),
        ],
        out_specs=pl.BlockSpec((1, sb, HID), lambda i, b: (b, i, 0)),
        out_shape=jax.ShapeDtypeStruct((bsz, seq_len, HID), jnp.float32),
    )(raw3, pos_emb, tt3, type_emb, gamma, beta)


def kernel(input_ids, token_type_ids, word_emb, pos_emb, type_emb, gamma, beta):
    bsz, seq_len = input_ids.shape
    total = bsz * seq_len
    sb = 512
    ids = input_ids.reshape(total).astype(jnp.int32)
    gather = _build_gather(total, nw=32, ch=32)
    raw = gather(ids, word_emb)
    raw3 = raw.reshape(bsz, seq_len, HID)
    tt3 = jnp.transpose(
        token_type_ids.astype(jnp.int32).reshape(bsz, seq_len // sb, sb),
        (1, 0, 2)).reshape(seq_len // sb, bsz, 1, sb)
    emb = _ln_tc(raw3, pos_emb, tt3, type_emb, gamma.reshape(1, HID),
                 beta.reshape(1, HID), bsz, seq_len, sb)
    return (emb, raw3)


# speedup vs baseline: 1.0940x; 1.0940x over previous
import functools

import jax
import jax.numpy as jnp
from jax import lax
from jax.experimental import pallas as pl
from jax.experimental.pallas import tpu as pltpu
from jax.experimental.pallas import tpu_sc as plsc

HID = 768
EPS = 1e-12


def _build_gather(total_tokens, nw, ch):
    tok_per_w = total_tokens // nw
    nchunks = tok_per_w // ch
    mesh = plsc.VectorSubcoreMesh(core_axis_name="c", subcore_axis_name="s")

    @functools.partial(
        pl.kernel,
        mesh=mesh,
        compiler_params=pltpu.CompilerParams(needs_layout_passes=False),
        out_type=jax.ShapeDtypeStruct((total_tokens, HID), jnp.float32),
        scratch_types=[
            pltpu.VMEM((tok_per_w,), jnp.int32),
            pltpu.VMEM((4, ch, HID), jnp.float32),
            pltpu.SemaphoreType.DMA,
            pltpu.SemaphoreType.DMA,
            pltpu.SemaphoreType.DMA,
            pltpu.SemaphoreType.DMA,
            pltpu.SemaphoreType.DMA,
            pltpu.SemaphoreType.DMA,
            pltpu.SemaphoreType.DMA,
            pltpu.SemaphoreType.DMA,
        ],
    )
    def gather_kernel(ids_hbm, wemb_hbm, raw_out, idx_v, a_v,
                      sg0, sg1, sg2, sg3, sw0, sw1, sw2, sw3):
        wid = lax.axis_index("s") * 2 + lax.axis_index("c")
        w0 = wid * tok_per_w
        sg = (sg0, sg1, sg2, sg3)
        sw = (sw0, sw1, sw2, sw3)

        pltpu.sync_copy(ids_hbm.at[pl.ds(w0, tok_per_w)], idx_v)

        def gather(cix, b):
            return pltpu.make_async_copy(
                wemb_hbm.at[idx_v.at[pl.ds(cix * ch, ch)]], a_v.at[b], sg[b])

        def write(cix, b):
            return pltpu.make_async_copy(
                a_v.at[b], raw_out.at[pl.ds(w0 + cix * ch, ch)], sw[b])

        for b in range(4):
            gather(b, b).start()

        def step(k, _):
            for b in range(4):
                cix = 4 * k + b
                gather(cix, b).wait()
                write(cix, b).start()

                @pl.when(k < nchunks // 4 - 1)
                def _():
                    write(cix, b).wait()
                    gather(cix + 4, b).start()

                @pl.when(k == nchunks // 4 - 1)
                def _():
                    write(cix, b).wait()
            return 0

        lax.fori_loop(0, nchunks // 4, step, 0)

    return gather_kernel


def _ln_block(raw_ref, pos_ref, tt_ref, temb_ref, gamma_ref, beta_ref,
              emb_ref):
    posq = pos_ref[...] + temb_ref[0, :][None, :]
    td = (temb_ref[1, :] - temb_ref[0, :])[None, None, :]
    ttf = tt_ref[0].astype(jnp.float32)
    x = (raw_ref[...] + posq[None, :, :]
         + ttf[:, :, None] * td)
    mean = jnp.mean(x, axis=-1, keepdims=True)
    xc = x - mean
    var = jnp.mean(xc * xc, axis=-1, keepdims=True)
    y = xc * lax.rsqrt(var + EPS)
    emb_ref[...] = y * gamma_ref[0][None, None, :] + beta_ref[0][None, None, :]


def _ln_tc(raw3, pos_emb, tt3, type_emb, gamma, beta, bsz, seq_len, sb):
    nblk = seq_len // sb
    return pl.pallas_call(
        _ln_block,
        grid=(nblk,),
        in_specs=[
            pl.BlockSpec((bsz, sb, HID), lambda i: (0, i, 0)),
            pl.BlockSpec((sb, HID), lambda i: (i, 0)),
            pl.BlockSpec((1, bsz, sb), lambda i: (i, 0, 0)),
            pl.BlockSpec((2, HID), lambda i: (0, 0)),
            pl.BlockSpec((1, HID), lambda i: (0, 0)),
            pl.BlockSpec((1, HID), lambda i: (0, 0)),
        ],
        out_specs=pl.BlockSpec((bsz, sb, HID), lambda i: (0, i, 0)),
        out_shape=jax.ShapeDtypeStruct((bsz, seq_len, HID), jnp.float32),
    )(raw3, pos_emb, tt3, type_emb, gamma, beta)


def kernel(input_ids, token_type_ids, word_emb, pos_emb, type_emb, gamma, beta):
    bsz, seq_len = input_ids.shape
    total = bsz * seq_len
    sb = 512
    ids = input_ids.reshape(total).astype(jnp.int32)
    gather = _build_gather(total, nw=32, ch=32)
    raw = gather(ids, word_emb)
    raw3 = raw.reshape(bsz, seq_len, HID)
    tt3 = jnp.transpose(
        token_type_ids.astype(jnp.int32).reshape(bsz, seq_len // sb, sb),
        (1, 0, 2))
    emb = _ln_tc(raw3, pos_emb, tt3, type_emb, gamma.reshape(1, HID),
                 beta.reshape(1, HID), bsz, seq_len, sb)
    return (emb, raw3)
